# baseline (speedup 1.0000x reference)
import functools

import jax
import jax.numpy as jnp
from jax import lax
from jax.experimental import pallas as pl
from jax.experimental.pallas import tpu as pltpu
from jax.experimental.pallas import tpu_sc as plsc

VOCAB = 1000000
EMBED = 64
OUT = 100000
B = 1024
L = 200

NC = 2
NS = 16
NW = NC * NS
BPW = B // NW
IPW = BPW * L
CH1, CH2 = 128, 72


def _sc_pool_body(idx_hbm, table_hbm, out_hbm, idx_v, rows_v, acc_v, sem):
    wid = lax.axis_index("s") * NC + lax.axis_index("c")
    pltpu.sync_copy(idx_hbm.at[pl.ds(wid * BPW, BPW)], idx_v)
    for i in range(BPW):
        g1 = pltpu.async_copy(
            table_hbm.at[idx_v.at[i, pl.ds(0, CH1)]],
            rows_v.at[pl.ds(0, CH1)], sem)
        g2 = pltpu.async_copy(
            table_hbm.at[idx_v.at[i, pl.ds(CH1, CH2)]],
            rows_v.at[pl.ds(CH1, CH2)], sem)
        g1.wait()
        g2.wait()

        def body(j, carry):
            a0, a1, a2, a3 = carry
            a0 = a0 + rows_v[j, pl.ds(0, 16)]
            a1 = a1 + rows_v[j, pl.ds(16, 16)]
            a2 = a2 + rows_v[j, pl.ds(32, 16)]
            a3 = a3 + rows_v[j, pl.ds(48, 16)]
            return a0, a1, a2, a3

        z = jnp.zeros((16,), jnp.float32)
        a0, a1, a2, a3 = lax.fori_loop(0, L, body, (z, z, z, z))
        acc_v[i, pl.ds(0, 16)] = a0
        acc_v[i, pl.ds(16, 16)] = a1
        acc_v[i, pl.ds(32, 16)] = a2
        acc_v[i, pl.ds(48, 16)] = a3
    pltpu.sync_copy(acc_v, out_hbm.at[pl.ds(wid * BPW, BPW)])


_sc_pool = functools.partial(
    pl.kernel,
    mesh=plsc.VectorSubcoreMesh(core_axis_name="c", subcore_axis_name="s"),
    out_type=jax.ShapeDtypeStruct((B, EMBED), jnp.float32),
    scratch_types=[
        pltpu.VMEM((BPW, L), jnp.int32),
        pltpu.VMEM((L, EMBED), jnp.float32),
        pltpu.VMEM((BPW, EMBED), jnp.float32),
        pltpu.SemaphoreType.DMA,
    ],
    compiler_params=pltpu.CompilerParams(use_tc_tiling_on_sc=False),
)(_sc_pool_body)


BLK = 2048
NBLK = (OUT + BLK - 1) // BLK


def _mm_body(p_ref, w_ref, b_ref, o_ref):
    o_ref[:] = lax.dot_general(
        p_ref[:], w_ref[:], (((1,), (1,)), ((), ())),
        preferred_element_type=jnp.float32) + b_ref[:]


def _matmul(pooled, W, b2):
    return pl.pallas_call(
        _mm_body,
        grid=(NBLK,),
        in_specs=[
            pl.BlockSpec((B, EMBED), lambda j: (0, 0)),
            pl.BlockSpec((BLK, EMBED), lambda j: (j, 0)),
            pl.BlockSpec((1, BLK), lambda j: (0, j)),
        ],
        out_specs=pl.BlockSpec((B, BLK), lambda j: (0, j)),
        out_shape=jax.ShapeDtypeStruct((B, OUT), jnp.float32),
    )(pooled, W, b2)


def kernel(inputs, table, W, b):
    pooled = _sc_pool(inputs.astype(jnp.int32), table)
    return _matmul(pooled, W, b.reshape(1, OUT))


# speedup vs baseline: 1.3141x; 1.3141x over previous
import functools

import jax
import jax.numpy as jnp
from jax import lax
from jax.experimental import pallas as pl
from jax.experimental.pallas import tpu as pltpu
from jax.experimental.pallas import tpu_sc as plsc

VOCAB = 1000000
EMBED = 64
OUT = 100000
B = 1024
L = 200

NC = 2
NS = 16
NW = NC * NS
BPW = B // NW
CH1, CH2 = 128, 72
PAIRS = VOCAB // 2


def _sc_pool_body(idx_hbm, ne_hbm, table_hbm, out_hbm, idx_v, ne_v, rows_v,
                  acc_v, sem):
    wid = lax.axis_index("s") * NC + lax.axis_index("c")
    base = wid * BPW
    pltpu.sync_copy(idx_hbm.at[pl.ds(base, BPW)], idx_v)
    pltpu.sync_copy(ne_hbm.at[pl.ds(base, BPW)], ne_v)
    lanes = lax.iota(jnp.int32, 16)
    for i in range(BPW):
        g1 = pltpu.async_copy(
            table_hbm.at[idx_v.at[i, pl.ds(0, CH1)]],
            rows_v.at[pl.ds(0, CH1)], sem)
        g2 = pltpu.async_copy(
            table_hbm.at[idx_v.at[i, pl.ds(CH1, CH2)]],
            rows_v.at[pl.ds(CH1, CH2)], sem)
        g1.wait()
        g2.wait()

        nvec = ne_v[pl.ds((i // 16) * 16, 16)]
        n_e = jnp.sum(jnp.where(lanes == (i % 16), nvec, 0))

        def lo_body(j, carry):
            a0, a1, a2, a3 = carry
            a0 = a0 + rows_v[j, pl.ds(0, 16)]
            a1 = a1 + rows_v[j, pl.ds(16, 16)]
            a2 = a2 + rows_v[j, pl.ds(32, 16)]
            a3 = a3 + rows_v[j, pl.ds(48, 16)]
            return a0, a1, a2, a3

        def hi_body(j, carry):
            a0, a1, a2, a3 = carry
            a0 = a0 + rows_v[j, pl.ds(64, 16)]
            a1 = a1 + rows_v[j, pl.ds(80, 16)]
            a2 = a2 + rows_v[j, pl.ds(96, 16)]
            a3 = a3 + rows_v[j, pl.ds(112, 16)]
            return a0, a1, a2, a3

        z = jnp.zeros((16,), jnp.float32)
        carry = lax.fori_loop(0, n_e, lo_body, (z, z, z, z))
        a0, a1, a2, a3 = lax.fori_loop(n_e, L, hi_body, carry)
        acc_v[i, pl.ds(0, 16)] = a0
        acc_v[i, pl.ds(16, 16)] = a1
        acc_v[i, pl.ds(32, 16)] = a2
        acc_v[i, pl.ds(48, 16)] = a3
    pltpu.sync_copy(acc_v, out_hbm.at[pl.ds(base, BPW)])


_sc_pool = functools.partial(
    pl.kernel,
    mesh=plsc.VectorSubcoreMesh(core_axis_name="c", subcore_axis_name="s"),
    out_type=jax.ShapeDtypeStruct((B, EMBED), jnp.float32),
    scratch_types=[
        pltpu.VMEM((BPW, L), jnp.int32),
        pltpu.VMEM((BPW,), jnp.int32),
        pltpu.VMEM((L, 2 * EMBED), jnp.float32),
        pltpu.VMEM((BPW, EMBED), jnp.float32),
        pltpu.SemaphoreType.DMA,
    ],
    compiler_params=pltpu.CompilerParams(needs_layout_passes=False),
)(_sc_pool_body)


BLK = 2048
NBLK = (OUT + BLK - 1) // BLK


def _mm_body(wt_ref, p_ref, b_ref, o_ref):
    o_ref[:] = lax.dot_general(
        wt_ref[:], p_ref[:], (((0,), (1,)), ((), ())),
        preferred_element_type=jnp.float32) + b_ref[:]


def _matmul_t(Wt, pooled, b2):
    return pl.pallas_call(
        _mm_body,
        grid=(NBLK,),
        in_specs=[
            pl.BlockSpec((EMBED, BLK), lambda j: (0, j)),
            pl.BlockSpec((B, EMBED), lambda j: (0, 0)),
            pl.BlockSpec((BLK, 1), lambda j: (j, 0)),
        ],
        out_specs=pl.BlockSpec((BLK, B), lambda j: (j, 0)),
        out_shape=jax.ShapeDtypeStruct((OUT, B), jnp.float32),
    )(Wt, pooled, b2)


def kernel(inputs, table, W, b):
    idx = inputs.astype(jnp.int32)
    parity = idx & 1
    order = jnp.argsort(parity, axis=1, stable=True)
    idx2 = jnp.take_along_axis(idx, order, axis=1) >> 1
    ne = (L - parity.sum(axis=1)).astype(jnp.int32)
    table128 = table.reshape(PAIRS, 2 * EMBED)
    pooled = _sc_pool(idx2, ne, table128)
    out_t = _matmul_t(W.T, pooled, b.reshape(OUT, 1))
    return out_t.T


# speedup vs baseline: 1.9872x; 1.5122x over previous
import functools

import jax
import jax.numpy as jnp
from jax import lax
from jax.experimental import pallas as pl
from jax.experimental.pallas import tpu as pltpu
from jax.experimental.pallas import tpu_sc as plsc

VOCAB = 1000000
EMBED = 64
OUT = 100000
B = 1024
L = 200

NC = 2
NS = 16
NW = NC * NS
BPW = B // NW
NG = L // 16
REM = L - NG * 16


def _sc_pool_body(idx_hbm, table_hbm, out_hbm, idx_v, rows_v, acc_v, sem):
    wid = lax.axis_index("s") * NC + lax.axis_index("c")
    base = wid * BPW
    pltpu.sync_copy(idx_hbm.at[pl.ds(base, BPW)], idx_v)
    lanes = lax.iota(jnp.int32, 16)

    def extract(vec, l):
        return jnp.sum(jnp.where(lanes == l, vec, 0))

    def row_body(i, carry_unused):
        def fire_group(g, _):
            vec = idx_v[i, pl.ds(g * 16, 16)]
            for l in range(16):
                r = extract(vec, l)
                pltpu.async_copy(
                    table_hbm.at[pl.ds(r, 1)],
                    rows_v.at[pl.ds(g * 16 + l, 1)], sem)
            return 0

        lax.fori_loop(0, NG, fire_group, 0)
        vec = idx_v[i, pl.ds(L - 16, 16)]
        for l in range(16 - REM, 16):
            r = extract(vec, l)
            pltpu.async_copy(
                table_hbm.at[pl.ds(r, 1)],
                rows_v.at[pl.ds(L - 16 + l, 1)], sem)
        pltpu.make_async_copy(table_hbm.at[pl.ds(0, L)], rows_v, sem).wait()

        def acc_body(j, carry):
            a0, a1, a2, a3 = carry
            a0 = a0 + rows_v[j, pl.ds(0, 16)]
            a1 = a1 + rows_v[j, pl.ds(16, 16)]
            a2 = a2 + rows_v[j, pl.ds(32, 16)]
            a3 = a3 + rows_v[j, pl.ds(48, 16)]
            return a0, a1, a2, a3

        z = jnp.zeros((16,), jnp.float32)
        a0, a1, a2, a3 = lax.fori_loop(0, L, acc_body, (z, z, z, z))
        acc_v[i, pl.ds(0, 16)] = a0
        acc_v[i, pl.ds(16, 16)] = a1
        acc_v[i, pl.ds(32, 16)] = a2
        acc_v[i, pl.ds(48, 16)] = a3
        return 0

    lax.fori_loop(0, BPW, row_body, 0)
    pltpu.sync_copy(acc_v, out_hbm.at[pl.ds(base, BPW)])


_sc_pool = functools.partial(
    pl.kernel,
    mesh=plsc.VectorSubcoreMesh(core_axis_name="c", subcore_axis_name="s"),
    out_type=jax.ShapeDtypeStruct((B, EMBED), jnp.float32),
    scratch_types=[
        pltpu.VMEM((BPW, L), jnp.int32),
        pltpu.VMEM((L, EMBED), jnp.float32),
        pltpu.VMEM((BPW, EMBED), jnp.float32),
        pltpu.SemaphoreType.DMA,
    ],
    compiler_params=pltpu.CompilerParams(needs_layout_passes=False),
)(_sc_pool_body)


BLK = 2048
NBLK = (OUT + BLK - 1) // BLK


def _mm_body(wt_ref, p_ref, b_ref, o_ref):
    o_ref[:] = lax.dot_general(
        wt_ref[:], p_ref[:], (((0,), (1,)), ((), ())),
        preferred_element_type=jnp.float32) + b_ref[:]


def _matmul_t(Wt, pooled, b2):
    return pl.pallas_call(
        _mm_body,
        grid=(NBLK,),
        in_specs=[
            pl.BlockSpec((EMBED, BLK), lambda j: (0, j)),
            pl.BlockSpec((B, EMBED), lambda j: (0, 0)),
            pl.BlockSpec((BLK, 1), lambda j: (j, 0)),
        ],
        out_specs=pl.BlockSpec((BLK, B), lambda j: (j, 0)),
        out_shape=jax.ShapeDtypeStruct((OUT, B), jnp.float32),
    )(Wt, pooled, b2)


def kernel(inputs, table, W, b):
    pooled = _sc_pool(inputs.astype(jnp.int32), table)
    out_t = _matmul_t(W.T, pooled, b.reshape(OUT, 1))
    return out_t.T
